# Initial kernel scaffold; baseline (speedup 1.0000x reference)
import jax
import jax.numpy as jnp
from jax.experimental import pallas as pl


def kernel(x, neighbors, W, b):
    raise NotImplementedError("write your pallas kernel here")


# speedup vs baseline: 1.2941x; 1.2941x over previous
import functools

import jax
import jax.numpy as jnp
from jax import lax
from jax.experimental import pallas as pl
from jax.experimental.pallas import tpu as pltpu
from jax.experimental.pallas import tpu_sc as plsc

N = 10000
DEG = 32
D = 128

NC = 2
NS = 16
NW = NC * NS
CHUNK = 320
NPAD = NW * CHUNK
GSZ = 4
NGRP = CHUNK // GSZ
ROWS = GSZ * DEG
LANES = 16


def _mm_body(x_ref, wt_ref, b_ref, z_ref, zb_ref):
    z = jnp.dot(x_ref[...], wt_ref[...], preferred_element_type=jnp.float32)
    z_ref[...] = z
    zb_ref[...] = z + b_ref[...]


def _matmul(x_pad, wt, b):
    return pl.pallas_call(
        _mm_body,
        out_shape=(
            jax.ShapeDtypeStruct((NPAD, D), jnp.float32),
            jax.ShapeDtypeStruct((NPAD, D), jnp.float32),
        ),
    )(x_pad, wt, b.reshape(1, D))


_sc_mesh = plsc.VectorSubcoreMesh(
    core_axis_name="c", subcore_axis_name="s", num_cores=NC, num_subcores=NS
)


@functools.partial(
    pl.kernel,
    mesh=_sc_mesh,
    out_type=jax.ShapeDtypeStruct((NPAD, D), jnp.float32),
    scratch_types=[
        pltpu.VMEM((NGRP, ROWS), jnp.int32),
        pltpu.VMEM((ROWS, D), jnp.float32),
        pltpu.VMEM((ROWS, D), jnp.float32),
        pltpu.VMEM((CHUNK, D), jnp.float32),
        pltpu.SemaphoreType.DMA,
        pltpu.SemaphoreType.DMA,
    ],
)
def _sc_agg(z_hbm, zb_hbm, nbr_hbm, out_hbm, idx_v, buf0, buf1, acc_v, sem0, sem1):
    wid = lax.axis_index("s") * NC + lax.axis_index("c")
    base = wid * CHUNK

    pltpu.sync_copy(nbr_hbm.at[pl.ds(wid * NGRP, NGRP)], idx_v)
    pltpu.sync_copy(zb_hbm.at[pl.ds(base, CHUNK)], acc_v)

    pltpu.async_copy(z_hbm.at[idx_v.at[0]], buf0, sem0)
    pltpu.async_copy(z_hbm.at[idx_v.at[1]], buf1, sem1)

    def _sum_rows(buf, r0, cs):
        accs = [buf[r0 + j, cs] for j in range(4)]
        for r in range(4, DEG):
            accs[r % 4] = accs[r % 4] + buf[r0 + r, cs]
        return (accs[0] + accs[1]) + (accs[2] + accs[3])

    def _compute_group(buf, g):
        for k in range(GSZ):
            l = g * GSZ + k
            r0 = DEG * k
            for c in range(D // LANES):
                cs = pl.ds(c * LANES, LANES)
                s = _sum_rows(buf, r0, cs)
                res = acc_v[l, cs] + s * (1.0 / DEG)
                acc_v[l, cs] = jnp.maximum(res, 0.0)

    def _body(t, carry):
        ga = 2 * t
        pltpu.make_async_copy(z_hbm.at[idx_v.at[ga]], buf0, sem0).wait()
        _compute_group(buf0, ga)

        @pl.when(ga + 2 < NGRP)
        def _():
            pltpu.async_copy(z_hbm.at[idx_v.at[ga + 2]], buf0, sem0)

        gb = ga + 1
        pltpu.make_async_copy(z_hbm.at[idx_v.at[gb]], buf1, sem1).wait()
        _compute_group(buf1, gb)

        @pl.when(gb + 2 < NGRP)
        def _():
            pltpu.async_copy(z_hbm.at[idx_v.at[gb + 2]], buf1, sem1)

        return carry

    lax.fori_loop(0, NGRP // 2, _body, 0)

    pltpu.sync_copy(acc_v, out_hbm.at[pl.ds(base, CHUNK)])


def kernel(x, neighbors, W, b):
    x = x.astype(jnp.float32)
    nbr = neighbors.astype(jnp.int32)
    x_pad = jnp.pad(x, ((0, NPAD - N), (0, 0)))
    nbr_pad = jnp.pad(nbr, ((0, NPAD - N), (0, 0))).reshape(NW * NGRP, ROWS)
    z, zb = _matmul(x_pad, W.T, b)
    out = _sc_agg(z, zb, nbr_pad)
    return out[:N]


# speedup vs baseline: 4.6827x; 3.6184x over previous
import functools

import jax
import jax.numpy as jnp
from jax import lax
from jax.experimental import pallas as pl
from jax.experimental.pallas import tpu as pltpu
from jax.experimental.pallas import tpu_sc as plsc

N = 10000
DEG = 32
D = 128

NC = 2
NS = 16
NW = NC * NS
CHUNK = 320
NPAD = NW * CHUNK
GSZ = 2
NGRP = CHUNK // GSZ
ROWS = GSZ * DEG
BLK = 64
GPB = BLK // GSZ
NBLK = CHUNK // BLK
LANES = 16


def _mm_body(x_ref, wt_ref, b_ref, z_ref, zb_ref):
    z = jnp.dot(x_ref[...], wt_ref[...], preferred_element_type=jnp.float32)
    z_ref[...] = z
    zb_ref[...] = z + b_ref[...]


def _matmul(x_pad, wt, b):
    return pl.pallas_call(
        _mm_body,
        out_shape=(
            jax.ShapeDtypeStruct((NPAD, D), jnp.float32),
            jax.ShapeDtypeStruct((NPAD, D), jnp.float32),
        ),
    )(x_pad, wt, b.reshape(1, D))


_sc_mesh = plsc.VectorSubcoreMesh(
    core_axis_name="c", subcore_axis_name="s", num_cores=NC, num_subcores=NS
)


@functools.partial(
    pl.kernel,
    mesh=_sc_mesh,
    out_type=jax.ShapeDtypeStruct((NPAD, D), jnp.float32),
    scratch_types=[
        pltpu.VMEM((NGRP, ROWS), jnp.int32),
        pltpu.VMEM((ROWS, D), jnp.float32),
        pltpu.VMEM((ROWS, D), jnp.float32),
        pltpu.VMEM((BLK, D), jnp.float32),
        pltpu.VMEM_SHARED((NPAD, D), jnp.float32),
        pltpu.SemaphoreType.DMA,
        pltpu.SemaphoreType.DMA,
    ],
)
def _sc_agg(z_hbm, zb_hbm, nbr_hbm, out_hbm, idx_v, buf0, buf1, acc_v, zs, sem0, sem1):
    wid = lax.axis_index("s") * NC + lax.axis_index("c")
    base = wid * CHUNK
    sid = lax.axis_index("s")

    seg = NPAD // NS
    pltpu.sync_copy(z_hbm.at[pl.ds(sid * seg, seg)], zs.at[pl.ds(sid * seg, seg)])

    pltpu.sync_copy(nbr_hbm.at[pl.ds(wid * NGRP, NGRP)], idx_v)

    plsc.subcore_barrier()

    pltpu.async_copy(zs.at[idx_v.at[0]], buf0, sem0)
    pltpu.async_copy(zs.at[idx_v.at[1]], buf1, sem1)

    def _sum_rows(buf, r0, cs):
        accs = [buf[r0 + j, cs] for j in range(4)]
        for r in range(4, DEG):
            accs[r % 4] = accs[r % 4] + buf[r0 + r, cs]
        return (accs[0] + accs[1]) + (accs[2] + accs[3])

    def _compute_group(buf, l0):
        for k in range(GSZ):
            r0 = DEG * k
            for c in range(D // LANES):
                cs = pl.ds(c * LANES, LANES)
                s = _sum_rows(buf, r0, cs)
                res = acc_v[l0 + k, cs] + s * (1.0 / DEG)
                acc_v[l0 + k, cs] = jnp.maximum(res, 0.0)

    for blk in range(NBLK):
        pltpu.sync_copy(zb_hbm.at[pl.ds(base + blk * BLK, BLK)], acc_v)

        def _body(t, carry):
            ga = blk * GPB + 2 * t
            pltpu.make_async_copy(zs.at[idx_v.at[ga]], buf0, sem0).wait()
            _compute_group(buf0, GSZ * 2 * t)

            @pl.when(ga + 2 < NGRP)
            def _():
                pltpu.async_copy(zs.at[idx_v.at[ga + 2]], buf0, sem0)

            gb = ga + 1
            pltpu.make_async_copy(zs.at[idx_v.at[gb]], buf1, sem1).wait()
            _compute_group(buf1, GSZ * (2 * t + 1))

            @pl.when(gb + 2 < NGRP)
            def _():
                pltpu.async_copy(zs.at[idx_v.at[gb + 2]], buf1, sem1)

            return carry

        lax.fori_loop(0, GPB // 2, _body, 0)

        pltpu.sync_copy(acc_v, out_hbm.at[pl.ds(base + blk * BLK, BLK)])


def kernel(x, neighbors, W, b):
    x = x.astype(jnp.float32)
    nbr = neighbors.astype(jnp.int32)
    x_pad = jnp.pad(x, ((0, NPAD - N), (0, 0)))
    nbr_pad = jnp.pad(nbr, ((0, NPAD - N), (0, 0))).reshape(NW * NGRP, ROWS)
    z, zb = _matmul(x_pad, W.T, b)
    out = _sc_agg(z, zb, nbr_pad)
    return out[:N]


# speedup vs baseline: 5.8219x; 1.2433x over previous
import functools

import jax
import jax.numpy as jnp
from jax import lax
from jax.experimental import pallas as pl
from jax.experimental.pallas import tpu as pltpu
from jax.experimental.pallas import tpu_sc as plsc

N = 10000
DEG = 32
D = 128

NC = 2
NS = 16
NW = NC * NS
CHUNK = 320
NPAD = NW * CHUNK
GSZ = 2
NGRP = CHUNK // GSZ
ROWS = GSZ * DEG
BLK = 64
GPB = BLK // GSZ
NBLK = CHUNK // BLK
LANES = 16


def _mm_body(x_ref, wt_ref, b_ref, z_ref, zb_ref):
    z = jnp.dot(x_ref[...], wt_ref[...], preferred_element_type=jnp.float32)
    z_ref[...] = z * (1.0 / DEG)
    zb_ref[...] = z + b_ref[...]


def _matmul(x_pad, wt, b):
    return pl.pallas_call(
        _mm_body,
        out_shape=(
            jax.ShapeDtypeStruct((NPAD, D), jnp.float32),
            jax.ShapeDtypeStruct((NPAD, D), jnp.float32),
        ),
    )(x_pad, wt, b.reshape(1, D))


_sc_mesh = plsc.VectorSubcoreMesh(
    core_axis_name="c", subcore_axis_name="s", num_cores=NC, num_subcores=NS
)


@functools.partial(
    pl.kernel,
    mesh=_sc_mesh,
    out_type=jax.ShapeDtypeStruct((NPAD, D), jnp.float32),
    scratch_types=[
        pltpu.VMEM((NGRP, ROWS), jnp.int32),
        pltpu.VMEM((ROWS, D), jnp.float32),
        pltpu.VMEM((ROWS, D), jnp.float32),
        pltpu.VMEM((BLK, D), jnp.float32),
        pltpu.VMEM_SHARED((NPAD, D), jnp.float32),
        pltpu.SemaphoreType.DMA,
        pltpu.SemaphoreType.DMA,
    ],
)
def _sc_agg(z_hbm, zb_hbm, nbr_hbm, out_hbm, idx_v, buf0, buf1, acc_v, zs, sem0, sem1):
    wid = lax.axis_index("s") * NC + lax.axis_index("c")
    base = wid * CHUNK
    sid = lax.axis_index("s")

    seg = NPAD // NS
    pltpu.sync_copy(z_hbm.at[pl.ds(sid * seg, seg)], zs.at[pl.ds(sid * seg, seg)])

    pltpu.sync_copy(nbr_hbm.at[pl.ds(wid * NGRP, NGRP)], idx_v)

    plsc.subcore_barrier()

    pltpu.async_copy(zs.at[idx_v.at[0]], buf0, sem0)
    pltpu.async_copy(zs.at[idx_v.at[1]], buf1, sem1)

    NSL = D // LANES

    def _compute_group(buf, l0):
        for k in range(GSZ):
            r0 = DEG * k
            for half in range(2):
                c0 = half * (NSL // 2)
                accs = [
                    buf[r0, pl.ds((c0 + i) * LANES, LANES)]
                    for i in range(NSL // 2)
                ]
                for r in range(1, DEG):
                    for i in range(NSL // 2):
                        accs[i] = accs[i] + buf[r0 + r, pl.ds((c0 + i) * LANES, LANES)]
                for i in range(NSL // 2):
                    cs = pl.ds((c0 + i) * LANES, LANES)
                    res = acc_v[l0 + k, cs] + accs[i]
                    acc_v[l0 + k, cs] = jnp.maximum(res, 0.0)

    for blk in range(NBLK):
        pltpu.sync_copy(zb_hbm.at[pl.ds(base + blk * BLK, BLK)], acc_v)

        def _body(t, carry):
            ga = blk * GPB + 2 * t
            pltpu.make_async_copy(zs.at[idx_v.at[ga]], buf0, sem0).wait()
            _compute_group(buf0, GSZ * 2 * t)

            @pl.when(ga + 2 < NGRP)
            def _():
                pltpu.async_copy(zs.at[idx_v.at[ga + 2]], buf0, sem0)

            gb = ga + 1
            pltpu.make_async_copy(zs.at[idx_v.at[gb]], buf1, sem1).wait()
            _compute_group(buf1, GSZ * (2 * t + 1))

            @pl.when(gb + 2 < NGRP)
            def _():
                pltpu.async_copy(zs.at[idx_v.at[gb + 2]], buf1, sem1)

            return carry

        lax.fori_loop(0, GPB // 2, _body, 0)

        pltpu.sync_copy(acc_v, out_hbm.at[pl.ds(base + blk * BLK, BLK)])


def kernel(x, neighbors, W, b):
    x = x.astype(jnp.float32)
    nbr = neighbors.astype(jnp.int32)
    x_pad = jnp.pad(x, ((0, NPAD - N), (0, 0)))
    nbr_pad = jnp.pad(nbr, ((0, NPAD - N), (0, 0))).reshape(NW * NGRP, ROWS)
    z, zb = _matmul(x_pad, W.T, b)
    out = _sc_agg(z, zb, nbr_pad)
    return out[:N]


# speedup vs baseline: 5.9852x; 1.0281x over previous
import functools

import jax
import jax.numpy as jnp
from jax import lax
from jax.experimental import pallas as pl
from jax.experimental.pallas import tpu as pltpu
from jax.experimental.pallas import tpu_sc as plsc

N = 10000
DEG = 32
D = 128

NC = 2
NS = 16
NW = NC * NS
CHUNK = 320
NPAD = NW * CHUNK
GSZ = 2
NGRP = CHUNK // GSZ
ROWS = GSZ * DEG
BLK = 64
GPB = BLK // GSZ
NBLK = CHUNK // BLK
LANES = 16


def _mm_body(x_ref, wt_ref, b_ref, z_ref, zb_ref):
    z = jnp.dot(x_ref[...], wt_ref[...], preferred_element_type=jnp.float32)
    z_ref[...] = z * (1.0 / DEG)
    zb_ref[...] = z + b_ref[...]


def _matmul(x_pad, wt, b):
    return pl.pallas_call(
        _mm_body,
        out_shape=(
            jax.ShapeDtypeStruct((NPAD, D), jnp.float32),
            jax.ShapeDtypeStruct((NPAD, D), jnp.float32),
        ),
    )(x_pad, wt, b.reshape(1, D))


_sc_mesh = plsc.VectorSubcoreMesh(
    core_axis_name="c", subcore_axis_name="s", num_cores=NC, num_subcores=NS
)


@functools.partial(
    pl.kernel,
    mesh=_sc_mesh,
    out_type=jax.ShapeDtypeStruct((N, D), jnp.float32),
    scratch_types=[
        pltpu.VMEM((NGRP, ROWS), jnp.int32),
        pltpu.VMEM((ROWS, D), jnp.float32),
        pltpu.VMEM((ROWS, D), jnp.float32),
        pltpu.VMEM((BLK, D), jnp.float32),
        pltpu.VMEM_SHARED((NPAD, D), jnp.float32),
        pltpu.SemaphoreType.DMA,
        pltpu.SemaphoreType.DMA,
    ],
)
def _sc_agg(z_hbm, zb_hbm, nbr_hbm, out_hbm, idx_v, buf0, buf1, acc_v, zs, sem0, sem1):
    wid = lax.axis_index("s") * NC + lax.axis_index("c")
    base = wid * CHUNK
    sid = lax.axis_index("s")

    seg = NPAD // NS
    pltpu.sync_copy(z_hbm.at[pl.ds(sid * seg, seg)], zs.at[pl.ds(sid * seg, seg)])

    pltpu.sync_copy(nbr_hbm.at[pl.ds(wid * NGRP, NGRP)], idx_v)

    plsc.subcore_barrier()

    pltpu.async_copy(zs.at[idx_v.at[0]], buf0, sem0)
    pltpu.async_copy(zs.at[idx_v.at[1]], buf1, sem1)

    NSL = D // LANES

    def _compute_group(buf, l0):
        for k in range(GSZ):
            r0 = DEG * k
            for half in range(2):
                c0 = half * (NSL // 2)
                accs = [
                    buf[r0, pl.ds((c0 + i) * LANES, LANES)]
                    for i in range(NSL // 2)
                ]
                for r in range(1, DEG):
                    for i in range(NSL // 2):
                        accs[i] = accs[i] + buf[r0 + r, pl.ds((c0 + i) * LANES, LANES)]
                for i in range(NSL // 2):
                    cs = pl.ds((c0 + i) * LANES, LANES)
                    res = acc_v[l0 + k, cs] + accs[i]
                    acc_v[l0 + k, cs] = jnp.maximum(res, 0.0)

    for blk in range(NBLK):
        pltpu.sync_copy(zb_hbm.at[pl.ds(base + blk * BLK, BLK)], acc_v)

        def _body(t, carry):
            ga = blk * GPB + 2 * t
            pltpu.make_async_copy(zs.at[idx_v.at[ga]], buf0, sem0).wait()
            _compute_group(buf0, GSZ * 2 * t)

            @pl.when(ga + 2 < NGRP)
            def _():
                pltpu.async_copy(zs.at[idx_v.at[ga + 2]], buf0, sem0)

            gb = ga + 1
            pltpu.make_async_copy(zs.at[idx_v.at[gb]], buf1, sem1).wait()
            _compute_group(buf1, GSZ * (2 * t + 1))

            @pl.when(gb + 2 < NGRP)
            def _():
                pltpu.async_copy(zs.at[idx_v.at[gb + 2]], buf1, sem1)

            return carry

        lax.fori_loop(0, GPB // 2, _body, 0)

        if blk == 0:
            pltpu.sync_copy(acc_v, out_hbm.at[pl.ds(base + blk * BLK, BLK)])
        else:

            @pl.when(wid != NW - 1)
            def _():
                pltpu.sync_copy(acc_v, out_hbm.at[pl.ds(base + blk * BLK, BLK)])

            if blk == 1:
                last = N - (NW - 1) * CHUNK - BLK

                @pl.when(wid == NW - 1)
                def _():
                    pltpu.sync_copy(
                        acc_v.at[pl.ds(0, last)],
                        out_hbm.at[pl.ds(base + blk * BLK, last)],
                    )


def kernel(x, neighbors, W, b):
    x = x.astype(jnp.float32)
    nbr = neighbors.astype(jnp.int32)
    x_pad = jnp.pad(x, ((0, NPAD - N), (0, 0)))
    nbr_pad = jnp.pad(nbr, ((0, NPAD - N), (0, 0))).reshape(NW * NGRP, ROWS)
    z, zb = _matmul(x_pad, W.T, b)
    return _sc_agg(z, zb, nbr_pad)


# speedup vs baseline: 6.0408x; 1.0093x over previous
import functools

import jax
import jax.numpy as jnp
from jax import lax
from jax.experimental import pallas as pl
from jax.experimental.pallas import tpu as pltpu
from jax.experimental.pallas import tpu_sc as plsc

N = 10000
DEG = 32
D = 128

NC = 2
NS = 16
NW = NC * NS
CHUNK = 320
NPAD = NW * CHUNK
GSZ = 2
NGRP = CHUNK // GSZ
ROWS = GSZ * DEG
BLK = 64
GPB = BLK // GSZ
NBLK = CHUNK // BLK
LANES = 16


def _mm_body(x_ref, wt_ref, b_ref, z_ref, zb_ref):
    z = jnp.dot(x_ref[...], wt_ref[...], preferred_element_type=jnp.float32)
    z_ref[...] = z * (1.0 / DEG)
    zb_ref[...] = z + b_ref[...]


def _matmul(x_pad, wt, b):
    return pl.pallas_call(
        _mm_body,
        out_shape=(
            jax.ShapeDtypeStruct((NPAD, D), jnp.float32),
            jax.ShapeDtypeStruct((NPAD, D), jnp.float32),
        ),
    )(x_pad, wt, b.reshape(1, D))


_sc_mesh = plsc.VectorSubcoreMesh(
    core_axis_name="c", subcore_axis_name="s", num_cores=NC, num_subcores=NS
)


@functools.partial(
    pl.kernel,
    mesh=_sc_mesh,
    out_type=jax.ShapeDtypeStruct((N, D), jnp.float32),
    scratch_types=[
        pltpu.VMEM((NGRP, ROWS), jnp.int32),
        pltpu.VMEM((ROWS, D), jnp.float32),
        pltpu.VMEM((ROWS, D), jnp.float32),
        pltpu.VMEM((BLK, D), jnp.float32),
        pltpu.VMEM_SHARED((NPAD, D), jnp.float32),
        pltpu.SemaphoreType.DMA,
        pltpu.SemaphoreType.DMA,
    ],
)
def _sc_agg(z_hbm, zb_hbm, nbr_hbm, out_hbm, idx_v, buf0, buf1, acc_v, zs, sem0, sem1):
    wid = lax.axis_index("s") * NC + lax.axis_index("c")
    base = wid * CHUNK
    sid = lax.axis_index("s")

    seg = NPAD // NS
    pltpu.async_copy(
        z_hbm.at[pl.ds(sid * seg, seg)], zs.at[pl.ds(sid * seg, seg)], sem0
    )
    pltpu.async_copy(nbr_hbm.at[pl.ds(wid * NGRP, NGRP)], idx_v, sem1)
    pltpu.make_async_copy(
        z_hbm.at[pl.ds(sid * seg, seg)], zs.at[pl.ds(sid * seg, seg)], sem0
    ).wait()
    pltpu.make_async_copy(nbr_hbm.at[pl.ds(wid * NGRP, NGRP)], idx_v, sem1).wait()

    plsc.subcore_barrier()

    pltpu.async_copy(zs.at[idx_v.at[0]], buf0, sem0)
    pltpu.async_copy(zs.at[idx_v.at[1]], buf1, sem1)

    NSL = D // LANES

    def _compute_group(buf, l0):
        for k in range(GSZ):
            r0 = DEG * k
            for half in range(2):
                c0 = half * (NSL // 2)
                accs = [
                    buf[r0, pl.ds((c0 + i) * LANES, LANES)]
                    for i in range(NSL // 2)
                ]
                for r in range(1, DEG):
                    for i in range(NSL // 2):
                        accs[i] = accs[i] + buf[r0 + r, pl.ds((c0 + i) * LANES, LANES)]
                for i in range(NSL // 2):
                    cs = pl.ds((c0 + i) * LANES, LANES)
                    res = acc_v[l0 + k, cs] + accs[i]
                    acc_v[l0 + k, cs] = jnp.maximum(res, 0.0)

    for blk in range(NBLK):
        pltpu.sync_copy(zb_hbm.at[pl.ds(base + blk * BLK, BLK)], acc_v)

        def _body(t, carry):
            ga = blk * GPB + 2 * t
            pltpu.make_async_copy(zs.at[idx_v.at[ga]], buf0, sem0).wait()
            _compute_group(buf0, GSZ * 2 * t)

            @pl.when(ga + 2 < NGRP)
            def _():
                pltpu.async_copy(zs.at[idx_v.at[ga + 2]], buf0, sem0)

            gb = ga + 1
            pltpu.make_async_copy(zs.at[idx_v.at[gb]], buf1, sem1).wait()
            _compute_group(buf1, GSZ * (2 * t + 1))

            @pl.when(gb + 2 < NGRP)
            def _():
                pltpu.async_copy(zs.at[idx_v.at[gb + 2]], buf1, sem1)

            return carry

        lax.fori_loop(0, GPB // 2, _body, 0)

        if blk == 0:
            pltpu.sync_copy(acc_v, out_hbm.at[pl.ds(base + blk * BLK, BLK)])
        else:

            @pl.when(wid != NW - 1)
            def _():
                pltpu.sync_copy(acc_v, out_hbm.at[pl.ds(base + blk * BLK, BLK)])

            if blk == 1:
                last = N - (NW - 1) * CHUNK - BLK

                @pl.when(wid == NW - 1)
                def _():
                    pltpu.sync_copy(
                        acc_v.at[pl.ds(0, last)],
                        out_hbm.at[pl.ds(base + blk * BLK, last)],
                    )


def kernel(x, neighbors, W, b):
    x = x.astype(jnp.float32)
    nbr = neighbors.astype(jnp.int32)
    x_pad = jnp.pad(x, ((0, NPAD - N), (0, 0)))
    nbr_pad = jnp.pad(nbr, ((0, NPAD - N), (0, 0))).reshape(NW * NGRP, ROWS)
    z, zb = _matmul(x_pad, W.T, b)
    return _sc_agg(z, zb, nbr_pad)
